# Initial kernel scaffold; baseline (speedup 1.0000x reference)
import jax
import jax.numpy as jnp
from jax.experimental import pallas as pl


def kernel(voxel_feats, edge_index, p2v_map, locs, clusters_pts, cluster_ids, params):
    raise NotImplementedError("write your pallas kernel here")


# speedup vs baseline: 7.3791x; 7.3791x over previous
import functools

import jax
import jax.numpy as jnp
from jax import lax
from jax.experimental import pallas as pl
from jax.experimental.pallas import tpu as pltpu
from jax.experimental.pallas import tpu_sc as plsc

_V = 50000
_N = 100000
_E = 800000
_P = 200000
_NC = 100
_SCALE = 50.0
_NW = 32
_VA = 50048
_VS = _VA // 16
_BV = 2000
_F32 = jnp.float32

_HI = jax.ShapeDtypeStruct


def _mesh():
    return plsc.VectorSubcoreMesh(core_axis_name="c", subcore_axis_name="s")


_SC_PARAMS = pltpu.CompilerParams(use_tc_tiling_on_sc=False)


def _pad_rows(x, pad_val, group=1024):
    n = x.shape[0]
    npad = (-n) % group
    if npad:
        x = jnp.concatenate([x, jnp.full((npad,), pad_val, jnp.int32)])
    return x.reshape(-1, 128)


def _sc_round1(vf16, src2d, dst2d, p2v2d, ones_row, zeros16):
    GE = src2d.shape[0] // 8
    GP = p2v2d.shape[0] // 8

    @functools.partial(
        pl.kernel,
        out_type=(_HI((2, _VA, 16), _F32), _HI((2, _VA, 16), _F32)),
        mesh=_mesh(),
        compiler_params=_SC_PARAMS,
        scratch_types=[
            pltpu.VMEM_SHARED((_VA, 16), _F32),
            pltpu.VMEM_SHARED((_VA, 16), _F32),
            pltpu.VMEM((8, 128), jnp.int32),
            pltpu.VMEM((8, 128), jnp.int32),
            pltpu.VMEM((128, 16), _F32),
            pltpu.VMEM((128, 16), _F32),
        ],
    )
    def k(vf_h, src_h, dst_h, p2v_h, ones_h, z_h, outA, outW,
          accA, accW, sidx, didx, rows, onesv):
        c = lax.axis_index("c")
        s = lax.axis_index("s")
        wid = c * 16 + s
        off = pl.multiple_of(s * _VS, 8)
        pltpu.sync_copy(z_h.at[pl.ds(off, _VS)], accA.at[pl.ds(off, _VS)])
        pltpu.sync_copy(z_h.at[pl.ds(off, _VS)], accW.at[pl.ds(off, _VS)])
        pltpu.sync_copy(ones_h, onesv)
        plsc.subcore_barrier()

        @pl.loop(wid * GE // _NW, (wid + 1) * GE // _NW)
        def _(g):
            r = pl.multiple_of(g * 8, 8)
            pltpu.sync_copy(src_h.at[pl.ds(r, 8)], sidx)
            pltpu.sync_copy(dst_h.at[pl.ds(r, 8)], didx)
            for j in range(8):
                pltpu.sync_copy(vf_h.at[sidx.at[j]], rows)
                pltpu.sync_copy(rows, accA.at[didx.at[j]], add=True)

        @pl.loop(wid * GP // _NW, (wid + 1) * GP // _NW)
        def _(g):
            r = pl.multiple_of(g * 8, 8)
            pltpu.sync_copy(p2v_h.at[pl.ds(r, 8)], sidx)
            for j in range(8):
                pltpu.sync_copy(onesv, accW.at[sidx.at[j]], add=True)

        plsc.subcore_barrier()
        pltpu.sync_copy(accA.at[pl.ds(off, _VS)], outA.at[c, pl.ds(off, _VS)])
        pltpu.sync_copy(accW.at[pl.ds(off, _VS)], outW.at[c, pl.ds(off, _VS)])

    return k(vf16, src2d, dst2d, p2v2d, ones_row, zeros16)


def _sc_agg32(h, src2d, dst2d, zeros32):
    GE = src2d.shape[0] // 8

    @functools.partial(
        pl.kernel,
        out_type=_HI((2, _VA, 32), _F32),
        mesh=_mesh(),
        compiler_params=_SC_PARAMS,
        scratch_types=[
            pltpu.VMEM_SHARED((_VA, 32), _F32),
            pltpu.VMEM((8, 128), jnp.int32),
            pltpu.VMEM((8, 128), jnp.int32),
            pltpu.VMEM((128, 32), _F32),
        ],
    )
    def k(h_h, src_h, dst_h, z_h, out, acc, sidx, didx, rows):
        c = lax.axis_index("c")
        s = lax.axis_index("s")
        wid = c * 16 + s
        off = pl.multiple_of(s * _VS, 8)
        pltpu.sync_copy(z_h.at[pl.ds(off, _VS)], acc.at[pl.ds(off, _VS)])
        plsc.subcore_barrier()

        @pl.loop(wid * GE // _NW, (wid + 1) * GE // _NW)
        def _(g):
            r = pl.multiple_of(g * 8, 8)
            pltpu.sync_copy(src_h.at[pl.ds(r, 8)], sidx)
            pltpu.sync_copy(dst_h.at[pl.ds(r, 8)], didx)
            for j in range(8):
                pltpu.sync_copy(h_h.at[sidx.at[j]], rows)
                pltpu.sync_copy(rows, acc.at[didx.at[j]], add=True)

        plsc.subcore_barrier()
        pltpu.sync_copy(acc.at[pl.ds(off, _VS)], out.at[c, pl.ds(off, _VS)])

    return k(h, src2d, dst2d, zeros32)


def _sc_gather_points(packed, p2v2d):
    RP = p2v2d.shape[0]
    GP = RP // 8
    NP = RP * 128

    @functools.partial(
        pl.kernel,
        out_type=_HI((NP, 24), _F32),
        mesh=_mesh(),
        compiler_params=_SC_PARAMS,
        scratch_types=[
            pltpu.VMEM((8, 128), jnp.int32),
            pltpu.VMEM((128, 24), _F32),
        ],
    )
    def k(tab_h, p2v_h, out, pidx, rows):
        c = lax.axis_index("c")
        s = lax.axis_index("s")
        wid = c * 16 + s

        @pl.loop(wid * GP // _NW, (wid + 1) * GP // _NW)
        def _(g):
            r = pl.multiple_of(g * 8, 8)
            pltpu.sync_copy(p2v_h.at[pl.ds(r, 8)], pidx)
            for j in range(8):
                pltpu.sync_copy(tab_h.at[pidx.at[j]], rows)
                o = pl.multiple_of((r + j) * 128, 128)
                pltpu.sync_copy(rows, out.at[pl.ds(o, 128)])

    return k(packed, p2v2d)


def _sc_cluster(locs16, p2v1, hs, cp2d, cid2d, geo_init, pool_init):
    GC = cp2d.shape[0] // 8

    @functools.partial(
        pl.kernel,
        out_type=(_HI((_NW, _NC + 1, 16), _F32),
                  _HI((_NW, _NC + 1, 32), _F32)),
        mesh=_mesh(),
        compiler_params=_SC_PARAMS,
        scratch_types=[
            pltpu.VMEM((8, 128), jnp.int32),
            pltpu.VMEM((128,), jnp.int32),
            pltpu.VMEM((128, 16), _F32),
            pltpu.VMEM((128, 32), _F32),
            pltpu.VMEM((_NC + 1, 16), _F32),
            pltpu.VMEM((_NC + 1, 32), _F32),
            pltpu.VMEM((8, 128), jnp.int32),
        ],
    )
    def k(locs_h, p2v_h, hs_h, cp_h, cid_h, gi_h, pi_h, out_geo, out_pool,
          cpidx, qv, lrows, hrows, geo_acc, pool_acc, cid_v):
        c = lax.axis_index("c")
        s = lax.axis_index("s")
        wid = c * 16 + s
        pltpu.sync_copy(gi_h, geo_acc)
        pltpu.sync_copy(pi_h, pool_acc)

        lane = lax.iota(jnp.int32, 16)
        m_sum = lane < 4
        m_min = jnp.logical_and(lane >= 4, lane < 8)

        @pl.loop(wid * GC // _NW, (wid + 1) * GC // _NW)
        def _(g):
            r = pl.multiple_of(g * 8, 8)
            pltpu.sync_copy(cp_h.at[pl.ds(r, 8)], cpidx)
            pltpu.sync_copy(cid_h.at[pl.ds(r, 8)], cid_v)
            for j in range(8):
                pltpu.sync_copy(locs_h.at[cpidx.at[j]], lrows)
                pltpu.sync_copy(p2v_h.at[cpidx.at[j]], qv)
                pltpu.sync_copy(hs_h.at[qv], hrows)

                @pl.loop(0, 8)
                def _(kk):
                    cidv = cid_v[j, pl.ds(kk * 16, 16)]
                    for jj in range(16):
                        cid = cidv[jj]
                        e = kk * 16 + jj
                        g_ = geo_acc[cid]
                        row = lrows[e]
                        comb = jnp.where(m_sum, g_ + row,
                                         jnp.where(m_min,
                                                   jnp.minimum(g_, row),
                                                   jnp.maximum(g_, row)))
                        geo_acc[cid] = comb
                        p0 = pool_acc[cid, pl.ds(0, 16)]
                        pool_acc[cid, pl.ds(0, 16)] = jnp.maximum(
                            p0, hrows[e, pl.ds(0, 16)])
                        p1 = pool_acc[cid, pl.ds(16, 16)]
                        pool_acc[cid, pl.ds(16, 16)] = jnp.maximum(
                            p1, hrows[e, pl.ds(16, 16)])

        pltpu.sync_copy(geo_acc, out_geo.at[wid])
        pltpu.sync_copy(pool_acc, out_pool.at[wid])

    return k(locs16, p2v1, hs, cp2d, cid2d, geo_init, pool_init)


_DOT = functools.partial(jnp.dot, preferred_element_type=_F32,
                         precision=lax.Precision.HIGHEST)


def _tc_p1a(vf16, agg0, Ws16, Wn16):
    G = _V // _BV

    def body(vf_ref, a_ref, ws_ref, wn_ref, y_ref, st_ref):
        i = pl.program_id(0)
        a = a_ref[0] + a_ref[1]
        y = _DOT(vf_ref[...], ws_ref[...]) + _DOT(a, wn_ref[...])
        y_ref[...] = y
        part = jnp.stack([y.sum(axis=0), (y * y).sum(axis=0)])

        @pl.when(i == 0)
        def _():
            st_ref[...] = part

        @pl.when(i > 0)
        def _():
            st_ref[...] += part

    return pl.pallas_call(
        body,
        grid=(G,),
        in_specs=[
            pl.BlockSpec((_BV, 16), lambda i: (i, 0)),
            pl.BlockSpec((2, _BV, 16), lambda i: (0, i, 0)),
            pl.BlockSpec((16, 32), lambda i: (0, 0)),
            pl.BlockSpec((16, 32), lambda i: (0, 0)),
        ],
        out_specs=[
            pl.BlockSpec((_BV, 32), lambda i: (i, 0)),
            pl.BlockSpec((2, 32), lambda i: (0, 0)),
        ],
        out_shape=[_HI((_V, 32), _F32), _HI((2, 32), _F32)],
    )(vf16, agg0, Ws16, Wn16)


def _tc_p1(mpart, W):
    G = _V // _BV

    def body(m_ref, w_ref, y_ref, st_ref):
        i = pl.program_id(0)
        y = _DOT(m_ref[0] + m_ref[1], w_ref[...])
        y_ref[...] = y
        part = jnp.stack([y.sum(axis=0), (y * y).sum(axis=0)])

        @pl.when(i == 0)
        def _():
            st_ref[...] = part

        @pl.when(i > 0)
        def _():
            st_ref[...] += part

    return pl.pallas_call(
        body,
        grid=(G,),
        in_specs=[
            pl.BlockSpec((2, _BV, 32), lambda i: (0, i, 0)),
            pl.BlockSpec((32, 32), lambda i: (0, 0)),
        ],
        out_specs=[
            pl.BlockSpec((_BV, 32), lambda i: (i, 0)),
            pl.BlockSpec((2, 32), lambda i: (0, 0)),
        ],
        out_shape=[_HI((_V, 32), _F32), _HI((2, 32), _F32)],
    )(mpart, W)


def _bn_from_stats(st_ref, g_ref, b_ref, denom):
    mu = st_ref[0:1, :] / denom
    var = st_ref[1:2, :] / denom - mu * mu
    rstd = lax.rsqrt(var + 1e-4)
    return mu, rstd * g_ref[...], b_ref[...]


def _tc_p2a(y, st, g, b):
    G = _V // _BV

    def body(y_ref, st_ref, g_ref, b_ref, h_ref):
        mu, sg, bb = _bn_from_stats(st_ref, g_ref, b_ref, float(_V))
        h_ref[...] = jnp.maximum((y_ref[...] - mu) * sg + bb, 0.0)

    return pl.pallas_call(
        body,
        grid=(G,),
        in_specs=[
            pl.BlockSpec((_BV, 32), lambda i: (i, 0)),
            pl.BlockSpec((2, 32), lambda i: (0, 0)),
            pl.BlockSpec((1, 32), lambda i: (0, 0)),
            pl.BlockSpec((1, 32), lambda i: (0, 0)),
        ],
        out_specs=pl.BlockSpec((_BV, 32), lambda i: (i, 0)),
        out_shape=_HI((_V, 32), _F32),
    )(y, st, g, b)


def _tc_p2res(hprev, y, st, g, b):
    G = _V // _BV

    def body(h_ref, y_ref, st_ref, g_ref, b_ref, o_ref):
        mu, sg, bb = _bn_from_stats(st_ref, g_ref, b_ref, float(_V))
        o_ref[...] = jnp.maximum(h_ref[...] + (y_ref[...] - mu) * sg + bb, 0.0)

    return pl.pallas_call(
        body,
        grid=(G,),
        in_specs=[
            pl.BlockSpec((_BV, 32), lambda i: (i, 0)),
            pl.BlockSpec((_BV, 32), lambda i: (i, 0)),
            pl.BlockSpec((2, 32), lambda i: (0, 0)),
            pl.BlockSpec((1, 32), lambda i: (0, 0)),
            pl.BlockSpec((1, 32), lambda i: (0, 0)),
        ],
        out_specs=pl.BlockSpec((_BV, 32), lambda i: (i, 0)),
        out_shape=_HI((_V, 32), _F32),
    )(hprev, y, st, g, b)


def _tc_p2c(hprev, y, st, g, b, wcnt_parts, W_sem, b_sem, W_o1, b_o1, W_sc1):
    G = _V // _BV

    def body(h_ref, y_ref, st_ref, g_ref, b_ref, wc_ref, wsem_ref, bsem_ref,
             wo1_ref, bo1_ref, wsc_ref, z_ref, hs_ref, sem_ref, zw_ref):
        i = pl.program_id(0)
        mu, sg, bb = _bn_from_stats(st_ref, g_ref, b_ref, float(_V))
        h2 = jnp.maximum(h_ref[...] + (y_ref[...] - mu) * sg + bb, 0.0)
        z = _DOT(h2, wo1_ref[...]) + bo1_ref[...]
        z_ref[...] = z
        hs_ref[...] = _DOT(h2, wsc_ref[...])
        sem_ref[...] = _DOT(h2, wsem_ref[...]) + bsem_ref[...]
        w = (wc_ref[0, :, 0:1] + wc_ref[1, :, 0:1])
        part = jnp.stack([(w * z).sum(axis=0), (w * z * z).sum(axis=0)])

        @pl.when(i == 0)
        def _():
            zw_ref[...] = part

        @pl.when(i > 0)
        def _():
            zw_ref[...] += part

    return pl.pallas_call(
        body,
        grid=(G,),
        in_specs=[
            pl.BlockSpec((_BV, 32), lambda i: (i, 0)),
            pl.BlockSpec((_BV, 32), lambda i: (i, 0)),
            pl.BlockSpec((2, 32), lambda i: (0, 0)),
            pl.BlockSpec((1, 32), lambda i: (0, 0)),
            pl.BlockSpec((1, 32), lambda i: (0, 0)),
            pl.BlockSpec((2, _BV, 16), lambda i: (0, i, 0)),
            pl.BlockSpec((32, 20), lambda i: (0, 0)),
            pl.BlockSpec((1, 20), lambda i: (0, 0)),
            pl.BlockSpec((32, 32), lambda i: (0, 0)),
            pl.BlockSpec((1, 32), lambda i: (0, 0)),
            pl.BlockSpec((32, 32), lambda i: (0, 0)),
        ],
        out_specs=[
            pl.BlockSpec((_BV, 32), lambda i: (i, 0)),
            pl.BlockSpec((_BV, 32), lambda i: (i, 0)),
            pl.BlockSpec((_BV, 20), lambda i: (i, 0)),
            pl.BlockSpec((2, 32), lambda i: (0, 0)),
        ],
        out_shape=[_HI((_V, 32), _F32), _HI((_V, 32), _F32),
                   _HI((_V, 20), _F32), _HI((2, 32), _F32)],
    )(hprev, y, st, g, b, wcnt_parts, W_sem, b_sem, W_o1, b_o1, W_sc1)


def _tc_p3(z, zw, sem_vox, go, bo, Wo2p, bo2p):
    G = _V // _BV

    def body(z_ref, zw_ref, sem_ref, g_ref, b_ref, w_ref, b2_ref, o_ref):
        mu, sg, bb = _bn_from_stats(zw_ref, g_ref, b_ref, float(_N))
        t = jnp.maximum((z_ref[...] - mu) * sg + bb, 0.0)
        off = _DOT(t, w_ref[...]) + b2_ref[...]
        o_ref[...] = jnp.concatenate([sem_ref[...], off], axis=1)

    return pl.pallas_call(
        body,
        grid=(G,),
        in_specs=[
            pl.BlockSpec((_BV, 32), lambda i: (i, 0)),
            pl.BlockSpec((2, 32), lambda i: (0, 0)),
            pl.BlockSpec((_BV, 20), lambda i: (i, 0)),
            pl.BlockSpec((1, 32), lambda i: (0, 0)),
            pl.BlockSpec((1, 32), lambda i: (0, 0)),
            pl.BlockSpec((32, 4), lambda i: (0, 0)),
            pl.BlockSpec((1, 4), lambda i: (0, 0)),
        ],
        out_specs=pl.BlockSpec((_BV, 24), lambda i: (i, 0)),
        out_shape=_HI((_VA, 24), _F32),
    )(z, zw, sem_vox, go, bo, Wo2p, bo2p)


def _tc_p4(part_geo, part_pool, W_slin, b_slin):
    def body(g_ref, p_ref, w_ref, b_ref, sc_ref, ce_ref, sz_ref):
        geo = g_ref[...]
        sums = geo[:, :_NC, 0:4].sum(axis=0)
        mins = geo[:, :_NC, 4:7].min(axis=0)
        maxs = geo[:, :_NC, 8:11].max(axis=0)
        cnt = sums[:, 3:4]
        cmean = _SCALE * sums[:, 0:3] / jnp.maximum(cnt, 1.0)
        cmin = jnp.minimum(_SCALE * mins - cmean, 1e9)
        cmax = jnp.maximum(_SCALE * maxs - cmean, -1e9)
        sz_ref[...] = cmax - cmin
        ce_ref[...] = (cmax + cmin) / 2.0 + cmean
        rawmax = p_ref[...][:, :_NC, :].max(axis=0)
        pooled = jnp.where(cnt > 0.0, jnp.maximum(rawmax, 0.0), -1e9)
        sc_ref[...] = _DOT(pooled, w_ref[...]) + b_ref[...]

    return pl.pallas_call(
        body,
        out_shape=[_HI((_NC, 1), _F32), _HI((_NC, 3), _F32),
                   _HI((_NC, 3), _F32)],
    )(part_geo, part_pool, W_slin, b_slin)


def kernel(voxel_feats, edge_index, p2v_map, locs, clusters_pts, cluster_ids,
           params):
    f32 = _F32
    vf16 = jnp.pad(voxel_feats, ((0, 0), (0, 10)))
    src2d = _pad_rows(edge_index[0], 0)
    dst2d = _pad_rows(edge_index[1], _V)
    p2v2d = _pad_rows(p2v_map, _V)
    cp2d = _pad_rows(clusters_pts, 0)
    cid2d = _pad_rows(cluster_ids, _NC)
    locs16 = jnp.concatenate(
        [locs, jnp.ones((_N, 1), f32), locs, jnp.zeros((_N, 1), f32),
         locs, jnp.zeros((_N, 5), f32)], axis=1)
    ones_row = jnp.concatenate(
        [jnp.ones((128, 1), f32), jnp.zeros((128, 15), f32)], axis=1)
    zeros16 = jnp.zeros((_VA, 16), f32)
    zeros32 = jnp.zeros((_VA, 32), f32)
    lane16 = jnp.arange(16)
    geo_init = jnp.where(lane16[None, :] < 4, 0.0,
                         jnp.where(lane16[None, :] < 8, 1e9, -1e9)
                         ).astype(f32) * jnp.ones((_NC + 1, 1), f32)
    pool_init = jnp.full((_NC + 1, 32), -1e9, f32)
    Ws16 = jnp.pad(params["W_self"], ((0, 10), (0, 0)))
    Wn16 = jnp.pad(params["W_nbr"], ((0, 10), (0, 0)))
    Wo2p = jnp.pad(params["W_o2"], ((0, 0), (0, 1)))
    bo2p = jnp.pad(params["b_o2"], ((0, 1),)).reshape(1, 4)
    g0 = params["g0"].reshape(1, 32)
    b0 = params["b0"].reshape(1, 32)
    g1 = params["g1"].reshape(1, 32)
    b1 = params["b1"].reshape(1, 32)
    g2 = params["g2"].reshape(1, 32)
    b2 = params["b2"].reshape(1, 32)
    go = params["go"].reshape(1, 32)
    bo = params["bo"].reshape(1, 32)
    b_sem = params["b_sem"].reshape(1, 20)

    agg0, wcnt_parts = _sc_round1(vf16, src2d, dst2d, p2v2d, ones_row, zeros16)
    y0, st0 = _tc_p1a(vf16, agg0, Ws16, Wn16)
    h = _tc_p2a(y0, st0, g0, b0)

    m1 = _sc_agg32(h, src2d, dst2d, zeros32)
    y1, st1 = _tc_p1(m1, params["Wr1"])
    h = _tc_p2res(h, y1, st1, g1, b1)

    m2 = _sc_agg32(h, src2d, dst2d, zeros32)
    y2, st2 = _tc_p1(m2, params["Wr2"])
    z, hs, sem_vox, zw = _tc_p2c(h, y2, st2, g2, b2, wcnt_parts,
                                 params["W_sem"], b_sem,
                                 params["W_o1"], params["b_o1"].reshape(1, 32),
                                 params["W_sc1"])

    packed = _tc_p3(z, zw, sem_vox, go, bo, Wo2p, bo2p)
    pts = _sc_gather_points(packed, p2v2d)
    semantic_scores = pts[:_N, :20]
    pt_offsets = pts[:_N, 20:23]

    part_geo, part_pool = _sc_cluster(locs16, p2v_map, hs, cp2d, cid2d,
                                      geo_init, pool_init)
    scores, clusters_center, clusters_size = _tc_p4(
        part_geo, part_pool, params["W_slin"],
        params["b_slin"].reshape(1, 1))

    return semantic_scores, pt_offsets, scores, clusters_center, clusters_size
